# baseline (speedup 1.0000x reference)
import jax
import jax.numpy as jnp
from jax import lax
from jax.experimental import pallas as pl
from jax.experimental.pallas import tpu as pltpu
from jax.experimental.pallas import tpu_sc as plsc

N_FIELDS = 26
VOCAB = 100000
EMB = 32
BATCH = 16384

NC = 2
NS = 16
NW = NC * NS
B_PER_W = BATCH // NW
ROWS_PER_CHUNK = 4
CHUNK = ROWS_PER_CHUNK * N_FIELDS
NCHUNKS = B_PER_W // ROWS_PER_CHUNK
NBUF = 4
NGRP = NCHUNKS // NBUF
LANES = 16


def _tec_body(x_hbm, tbl_hbm, out_hbm, x_v, idx_v, gbuf, acc, ld_sem, g_sems):
  wid = lax.axis_index("s") * NC + lax.axis_index("c")

  pltpu.async_copy(x_hbm.at[pl.ds(wid * B_PER_W, B_PER_W)], x_v, ld_sem).wait()

  off0 = lax.iota(jnp.int32, LANES) * VOCAB
  off1 = off0 + 10 * VOCAB

  def _build_and_fire(ch, b):
    for r in range(ROWS_PER_CHUNK):
      row = ch * ROWS_PER_CHUNK + r
      idx_v[b, pl.ds(r * N_FIELDS, LANES)] = (
          x_v[row, pl.ds(0, LANES)] + off0)
      idx_v[b, pl.ds(r * N_FIELDS + 10, LANES)] = (
          x_v[row, pl.ds(10, LANES)] + off1)
    pltpu.async_copy(tbl_hbm.at[idx_v.at[b]], gbuf.at[pl.ds(b * CHUNK, CHUNK)],
                     g_sems.at[b])

  for b in range(NBUF):
    _build_and_fire(b, b)

  def _grp(g, c):
    for b in range(NBUF):
      ch = g * NBUF + b
      pltpu.make_async_copy(tbl_hbm.at[idx_v.at[b]],
                            gbuf.at[pl.ds(b * CHUNK, CHUNK)],
                            g_sems.at[b]).wait()

      for r in range(ROWS_PER_CHUNK):
        src = b * CHUNK + r * N_FIELDS
        p0 = [gbuf[src + i, pl.ds(0, LANES)] for i in range(4)]
        p1 = [gbuf[src + i, pl.ds(LANES, LANES)] for i in range(4)]
        for i in range(4, N_FIELDS):
          p0[i % 4] += gbuf[src + i, pl.ds(0, LANES)]
          p1[i % 4] += gbuf[src + i, pl.ds(LANES, LANES)]
        row = ch * ROWS_PER_CHUNK + r
        acc[row, pl.ds(0, LANES)] = (p0[0] + p0[1]) + (p0[2] + p0[3])
        acc[row, pl.ds(LANES, LANES)] = (p1[0] + p1[1]) + (p1[2] + p1[3])

      @pl.when(g < NGRP - 1)
      def _():
        _build_and_fire(ch + NBUF, b)

    return c

  lax.fori_loop(0, NGRP, _grp, 0, unroll=False)

  pltpu.async_copy(acc, out_hbm.at[pl.ds(wid * B_PER_W, B_PER_W)],
                   ld_sem).wait()


@jax.jit
def kernel(x, tables):
  tbl_flat = tables.reshape(N_FIELDS * VOCAB, EMB)

  mesh = plsc.VectorSubcoreMesh(core_axis_name="c", subcore_axis_name="s")
  f = pl.kernel(
      _tec_body,
      out_type=jax.ShapeDtypeStruct((BATCH, EMB), jnp.float32),
      mesh=mesh,
      compiler_params=pltpu.CompilerParams(use_tc_tiling_on_sc=False),
      scratch_types=[
          pltpu.VMEM((B_PER_W, N_FIELDS), jnp.int32),
          pltpu.VMEM((NBUF, CHUNK), jnp.int32),
          pltpu.VMEM((NBUF * CHUNK, EMB), jnp.float32),
          pltpu.VMEM((B_PER_W, EMB), jnp.float32),
          pltpu.SemaphoreType.DMA,
          pltpu.SemaphoreType.DMA((NBUF,)),
      ],
  )
  return f(x, tbl_flat)
